# Initial kernel scaffold; baseline (speedup 1.0000x reference)
import jax
import jax.numpy as jnp
from jax.experimental import pallas as pl


def kernel(x, edge_index, W1, b1, W2, b2):
    raise NotImplementedError("write your pallas kernel here")


# speedup vs baseline: 3.7013x; 3.7013x over previous
import functools

import jax
import jax.numpy as jnp
from jax import lax
from jax.experimental import pallas as pl
from jax.experimental.pallas import tpu as pltpu
from jax.experimental.pallas import tpu_sc as plsc

MIN_NORM = 1e-15
EPS = 4e-3

N_NODES = 10000
N_EDGES = 160000
D = 256
HALF = 128

NS = 16
CHUNK = 128
CHUNKS_PER_TILE = 79
E_PAD = NS * CHUNKS_PER_TILE * CHUNK
N_PAD = 10240
ROWS_PER_TILE = N_PAD // NS
ROW_CHUNK = 64
ROW_CHUNKS = ROWS_PER_TILE // ROW_CHUNK


def _norm(x):
    return jnp.clip(
        jnp.sqrt(jnp.sum(x * x, axis=-1, keepdims=True)), MIN_NORM, None)


def _artanh(x):
    x = jnp.clip(x, -1.0 + 1e-7, 1.0 - 1e-7)
    return 0.5 * jnp.log((1.0 + x) / (1.0 - x))


def _proj(x):
    norm = _norm(x)
    maxnorm = 1.0 - EPS
    return jnp.where(norm > maxnorm, x / norm * maxnorm, x)


def _expmap0(u):
    u_norm = _norm(u)
    return jnp.tanh(u_norm) * u / u_norm


def _logmap0(p):
    p_norm = _norm(p)
    return _artanh(p_norm) * p / p_norm


def _mobius_add(x, y):
    x2 = jnp.sum(x * x, axis=-1, keepdims=True)
    y2 = jnp.sum(y * y, axis=-1, keepdims=True)
    xy = jnp.sum(x * y, axis=-1, keepdims=True)
    num = (1.0 + 2.0 * xy + y2) * x + (1.0 - x2) * y
    denom = 1.0 + 2.0 * xy + x2 * y2
    return num / jnp.clip(denom, MIN_NORM, None)


def _hyp_linear(xh, w, b_row):
    x_norm = _norm(xh)
    mx = lax.dot_general(xh, w, (((1,), (1,)), ((), ())),
                         preferred_element_type=jnp.float32)
    mx_norm = _norm(mx)
    res_c = jnp.tanh(mx_norm / x_norm * _artanh(x_norm)) * mx / mx_norm
    cond = jnp.all(mx == 0, axis=-1, keepdims=True)
    res = jnp.where(cond, jnp.zeros_like(res_c), res_c)
    res = _proj(res)
    hyp_b = _proj(_expmap0(b_row))
    return _proj(_mobius_add(res, hyp_b))


def _post_agg(agg):
    h = _proj(_expmap0(agg))
    xt = jax.nn.relu(_logmap0(h))
    return _proj(_expmap0(xt))


def _pre1_body(x_ref, w_ref, b_ref, lo_ref, hi_ref):
    x = x_ref[...]
    xh = _proj(_expmap0(x))
    xt = _logmap0(_hyp_linear(xh, w_ref[...], b_ref[...]))
    lo_ref[...] = xt[:, :HALF]
    hi_ref[...] = xt[:, HALF:]


def _mid_body(lo_in, hi_in, w_ref, b_ref, lo_ref, hi_ref):
    agg = jnp.concatenate([lo_in[...], hi_in[...]], axis=1)
    u = _post_agg(agg)
    xt = _logmap0(_hyp_linear(u, w_ref[...], b_ref[...]))
    lo_ref[...] = xt[:, :HALF]
    hi_ref[...] = xt[:, HALF:]


def _final_body(lo_in, hi_in, out_ref):
    agg = jnp.concatenate([lo_in[...], hi_in[...]], axis=1)
    out_ref[...] = _post_agg(agg)


_BLK = 1000
_GRID = N_NODES // _BLK

_row_spec = pl.BlockSpec((_BLK, D), lambda i: (i, 0))
_half_spec = pl.BlockSpec((_BLK, HALF), lambda i: (i, 0))
_w_spec = pl.BlockSpec((D, D), lambda i: (0, 0))
_b_spec = pl.BlockSpec((1, D), lambda i: (0, 0))

_half_sds = jax.ShapeDtypeStruct((N_NODES, HALF), jnp.float32)

_pre1 = pl.pallas_call(
    _pre1_body,
    grid=(_GRID,),
    in_specs=[_row_spec, _w_spec, _b_spec],
    out_specs=[_half_spec, _half_spec],
    out_shape=[_half_sds, _half_sds],
)

_mid = pl.pallas_call(
    _mid_body,
    grid=(_GRID,),
    in_specs=[_half_spec, _half_spec, _w_spec, _b_spec],
    out_specs=[_half_spec, _half_spec],
    out_shape=[_half_sds, _half_sds],
)

_final = pl.pallas_call(
    _final_body,
    grid=(_GRID,),
    in_specs=[_half_spec, _half_spec],
    out_specs=_row_spec,
    out_shape=jax.ShapeDtypeStruct((N_NODES, D), jnp.float32),
)


def _seg_sum_body(lo_hbm, hi_hbm, src_hbm, dst_hbm, out_lo, out_hi,
                  accum, sidx, didx, rows, obuf, sem):
    c = lax.axis_index("c")
    s = lax.axis_index("s")

    zero = jnp.zeros((16,), jnp.float32)

    def zrow(i, _):
        def zcol(j, _):
            obuf[i, pl.ds(j * 16, 16)] = zero
            return 0
        return lax.fori_loop(0, HALF // 16, zcol, 0)

    lax.fori_loop(0, ROW_CHUNK, zrow, 0)
    for q in range(ROW_CHUNKS):
        pltpu.sync_copy(
            obuf, accum.at[pl.ds(s * ROWS_PER_TILE + q * ROW_CHUNK,
                                 ROW_CHUNK)])

    pltpu.sync_copy(src_hbm.at[s], sidx)
    pltpu.sync_copy(dst_hbm.at[s], didx)

    plsc.subcore_barrier()

    def run(table_hbm, out_hbm):
        def step(j, _):
            pltpu.async_copy(table_hbm.at[sidx.at[j]], rows, sem).wait()
            pltpu.sync_copy(rows, accum.at[didx.at[j]], add=True)
            return 0

        lax.fori_loop(0, CHUNKS_PER_TILE, step, 0)
        plsc.subcore_barrier()
        for q in range(ROW_CHUNKS):
            base = s * ROWS_PER_TILE + q * ROW_CHUNK
            pltpu.sync_copy(accum.at[pl.ds(base, ROW_CHUNK)], obuf)
            pltpu.sync_copy(obuf, out_hbm.at[pl.ds(base, ROW_CHUNK)])

    @pl.when(c == 0)
    def _():
        run(lo_hbm, out_lo)

    @pl.when(c == 1)
    def _():
        run(hi_hbm, out_hi)


_pad_sds = jax.ShapeDtypeStruct((N_PAD, HALF), jnp.float32)


@functools.cache
def _get_seg_sum():
    return functools.partial(
        pl.kernel,
        out_type=[_pad_sds, _pad_sds],
        mesh=plsc.VectorSubcoreMesh(core_axis_name="c",
                                    subcore_axis_name="s"),
        scratch_types=[
            pltpu.VMEM_SHARED((N_PAD, HALF), jnp.float32),
            pltpu.VMEM((CHUNKS_PER_TILE, CHUNK), jnp.int32),
            pltpu.VMEM((CHUNKS_PER_TILE, CHUNK), jnp.int32),
            pltpu.VMEM((CHUNK, HALF), jnp.float32),
            pltpu.VMEM((ROW_CHUNK, HALF), jnp.float32),
            pltpu.SemaphoreType.DMA,
        ],
    )(_seg_sum_body)


def kernel(x, edge_index, W1, b1, W2, b2):
    n_extra = E_PAD - N_EDGES
    src_pad = jnp.zeros((n_extra,), jnp.int32)
    dst_pad = N_NODES + (jnp.arange(n_extra, dtype=jnp.int32)
                         % (N_PAD - N_NODES))
    src = jnp.concatenate(
        [edge_index[0].astype(jnp.int32), src_pad]).reshape(
            NS, CHUNKS_PER_TILE, CHUNK)
    dst = jnp.concatenate(
        [edge_index[1].astype(jnp.int32), dst_pad]).reshape(
            NS, CHUNKS_PER_TILE, CHUNK)

    seg_sum = _get_seg_sum()
    lo1, hi1 = _pre1(x, W1, b1.reshape(1, D))
    alo1, ahi1 = seg_sum(lo1, hi1, src, dst)
    lo2, hi2 = _mid(alo1, ahi1, W2, b2.reshape(1, D))
    alo2, ahi2 = seg_sum(lo2, hi2, src, dst)
    return _final(alo2, ahi2)
